# baseline (speedup 1.0000x reference)
import jax
import jax.numpy as jnp
from jax.experimental import pallas as pl

AXON = 1000
NEURON = 10000
T_STEPS = 16
BETA = 0.95
THRESH = 1.0
B = 128

NT = 1024
BT = 16


def _matmul_body(x_ref, w_ref, m_ref, b_ref, cur_ref):
    wm = w_ref[...] * m_ref[...].astype(jnp.float32)
    cur_ref[...] = jax.lax.dot_general(
        x_ref[...], wm,
        dimension_numbers=(((1,), (1,)), ((), ())),
        preferred_element_type=jnp.float32,
    ) + b_ref[...]


def _lif_body(cur_ref, spk_ref, mem_ref):
    cur = cur_ref[...]
    mem = jnp.zeros_like(cur)
    for t in range(T_STEPS):
        reset = (mem > THRESH).astype(jnp.float32)
        mem = BETA * mem + cur - reset * THRESH
        spk_ref[t] = (mem > THRESH).astype(jnp.float32)
        mem_ref[t] = mem


def kernel(x, W, b, mask):
    b2 = b.reshape(1, NEURON)
    cur = pl.pallas_call(
        _matmul_body,
        grid=(pl.cdiv(NEURON, NT),),
        in_specs=[
            pl.BlockSpec((B, AXON), lambda i: (0, 0)),
            pl.BlockSpec((NT, AXON), lambda i: (i, 0)),
            pl.BlockSpec((NT, AXON), lambda i: (i, 0)),
            pl.BlockSpec((1, NT), lambda i: (0, i)),
        ],
        out_specs=pl.BlockSpec((B, NT), lambda i: (0, i)),
        out_shape=jax.ShapeDtypeStruct((B, NEURON), jnp.float32),
    )(x, W, mask, b2)

    spk, mem = pl.pallas_call(
        _lif_body,
        grid=(B // BT,),
        in_specs=[pl.BlockSpec((BT, NEURON), lambda i: (i, 0))],
        out_specs=[
            pl.BlockSpec((T_STEPS, BT, NEURON), lambda i: (0, i, 0)),
            pl.BlockSpec((T_STEPS, BT, NEURON), lambda i: (0, i, 0)),
        ],
        out_shape=[
            jax.ShapeDtypeStruct((T_STEPS, B, NEURON), jnp.float32),
            jax.ShapeDtypeStruct((T_STEPS, B, NEURON), jnp.float32),
        ],
    )(cur)
    return spk, mem
